# baseline (speedup 1.0000x reference)
import functools

import jax
import jax.numpy as jnp
from jax import lax
from jax.experimental import pallas as pl
from jax.experimental.pallas import tpu as pltpu
from jax.experimental.pallas import tpu_sc as plsc

D = 64
S = 200
B = 4096
NC = 2
NS = 16
NW = NC * NS
SCALE = 8.0

SEQ_PER_CHUNK = 2
CHUNK = SEQ_PER_CHUNK * S
SUB = 100
NSUB = CHUNK // SUB
NBUF = 4
ROWS_PER_W = (B * S) // NW
N_CHUNKS = ROWS_PER_W // CHUNK


def _encoder_body(x_hbm, tok_hbm, pos_hbm, out_hbm,
                  idx_v, rows_v, pe_v, gsem, osem, isem):
    wid = lax.axis_index("s") * NC + lax.axis_index("c")
    out_base = wid * ROWS_PER_W

    pltpu.sync_copy(pos_hbm, pe_v)

    def issue_gathers(c, b):
        for j in range(NSUB):
            pltpu.async_copy(
                tok_hbm.at[idx_v.at[b, j]],
                rows_v.at[b, pl.ds(j * SUB, SUB)],
                gsem.at[b],
            )

    def wait_gathers(b):
        for j in range(NSUB):
            pltpu.make_async_copy(
                tok_hbm.at[idx_v.at[b, j]],
                rows_v.at[b, pl.ds(j * SUB, SUB)],
                gsem.at[b],
            ).wait()

    def issue_idx_load(c, b):
        pltpu.async_copy(x_hbm.at[wid, c], idx_v.at[b], isem.at[b])

    def wait_idx_load(b):
        pltpu.make_async_copy(
            x_hbm.at[wid, 0], idx_v.at[b], isem.at[b]
        ).wait()

    def compute_chunk(b):
        @pl.loop(0, S)
        def _per_position(s):
            for d in range(D // 16):
                sl = pl.ds(d * 16, 16)
                pe_d = pe_v[s, sl]
                for j in range(SEQ_PER_CHUNK):
                    r = j * S + s
                    rows_v[b, r, sl] = rows_v[b, r, sl] * SCALE + pe_d

    def issue_scatter(c, b):
        pltpu.async_copy(
            rows_v.at[b],
            out_hbm.at[pl.ds(out_base + c * CHUNK, CHUNK)],
            osem.at[b],
        )

    def wait_scatter(c, b):
        pltpu.make_async_copy(
            rows_v.at[b],
            out_hbm.at[pl.ds(out_base + c * CHUNK, CHUNK)],
            osem.at[b],
        ).wait()

    for b in range(NBUF - 1):
        issue_idx_load(b, b)
        wait_idx_load(b)
        issue_gathers(b, b)
    issue_idx_load(NBUF - 1, NBUF - 1)

    @pl.loop(0, N_CHUNKS, step=NBUF)
    def _chunk_group(c0):
        for b in range(NBUF):
            c = c0 + b
            prev = (b - 1) % NBUF

            @pl.when(c > 0)
            def _():
                wait_scatter(c - 1, prev)

            @pl.when(c + NBUF - 1 < N_CHUNKS)
            def _():
                wait_idx_load(prev)
                issue_gathers(c + NBUF - 1, prev)

            wait_gathers(b)

            @pl.when(c + NBUF < N_CHUNKS)
            def _():
                issue_idx_load(c + NBUF, b)

            compute_chunk(b)
            issue_scatter(c, b)

    wait_scatter(N_CHUNKS - 1, (N_CHUNKS - 1) % NBUF)


@jax.jit
def _encoder(x_r, tok_table, pos_table):
    mesh = plsc.VectorSubcoreMesh(core_axis_name="c", subcore_axis_name="s")
    return pl.kernel(
        _encoder_body,
        out_type=jax.ShapeDtypeStruct((B * S, D), jnp.float32),
        mesh=mesh,
        compiler_params=pltpu.CompilerParams(use_tc_tiling_on_sc=False),
        scratch_types=[
            pltpu.VMEM((NBUF, NSUB, SUB), jnp.int32),
            pltpu.VMEM((NBUF, CHUNK, D), jnp.float32),
            pltpu.VMEM((S, D), jnp.float32),
            pltpu.SemaphoreType.DMA((NBUF,)),
            pltpu.SemaphoreType.DMA((NBUF,)),
            pltpu.SemaphoreType.DMA((NBUF,)),
        ],
    )(x_r, tok_table, pos_table)


def kernel(x, mask, tok_table, pos_table):
    del mask
    x_r = x.astype(jnp.int32).reshape(NW, N_CHUNKS, NSUB, SUB)
    out = _encoder(x_r, tok_table, pos_table)
    return out.reshape(B, S, D)
